# Initial kernel scaffold; baseline (speedup 1.0000x reference)
import jax
import jax.numpy as jnp
from jax.experimental import pallas as pl


def kernel(x, mask, Wg, Wqkv, Wff, bff):
    raise NotImplementedError("write your pallas kernel here")


# speedup vs baseline: 3.0458x; 3.0458x over previous
import functools

import numpy as np
import jax
import jax.numpy as jnp
from jax import lax
from jax.experimental import pallas as pl
from jax.experimental.pallas import tpu as pltpu
from jax.experimental.pallas import tpu_sc as plsc

E = 8
S = 2048
D = 1024
HD = D // E
RP = HD // 2
F3 = 3 * HD
NH = 16
SCALE = 1.0 / float(np.sqrt(D // NH))
BLK = 256
NB = S // BLK
NEG = -1e30


def _rotary_tables():
    freqs = (1.0 / (10000.0 ** (np.arange(0, RP, 2, dtype=np.float32) / np.float32(RP)))).astype(np.float32)
    ang = np.arange(S, dtype=np.float32)[:, None] * freqs[None, :]
    cos = np.cos(ang).astype(np.float32)
    sin = np.sin(ang).astype(np.float32)
    P1 = np.zeros((D,), dtype=np.int32)
    P2 = np.zeros((D,), dtype=np.int32)
    A = np.zeros((S, D), dtype=np.float32)
    B = np.zeros((S, D), dtype=np.float32)
    for c in range(E):
        base = c * HD
        for o in range(HD):
            oc = base + o
            if o < RP:
                j = o // 2
                P1[oc] = base + RP + 2 * j
                P2[oc] = base + RP + 2 * j + 1
                if o % 2 == 0:
                    A[:, oc] = cos[:, j]
                    B[:, oc] = -sin[:, j]
                else:
                    A[:, oc] = sin[:, j]
                    B[:, oc] = cos[:, j]
            else:
                P1[oc] = base + (o - RP)
                P2[oc] = base
                A[:, oc] = 1.0
                B[:, oc] = 0.0
    return P1, P2, A, B


_P1, _P2, _AC, _BC = _rotary_tables()


def _gate_body(xp1_ref, xp2_ref, ac_ref, bc_ref, wg_ref, x2_ref, eid_ref, gate_ref):
    x2 = xp1_ref[...] * ac_ref[...] + xp2_ref[...] * bc_ref[...]
    x2_ref[...] = x2
    logits = jnp.dot(x2, wg_ref[...], preferred_element_type=jnp.float32)
    m = jnp.max(logits, axis=1, keepdims=True)
    p = jnp.exp(logits - m)
    ssum = jnp.sum(p, axis=1, keepdims=True)
    probs = p / ssum
    pmax = jnp.max(probs, axis=1, keepdims=True)
    ids = lax.broadcasted_iota(jnp.int32, (BLK, E), 1).astype(jnp.float32)
    eid_ref[...] = jnp.min(jnp.where(probs >= pmax, ids, float(E)), axis=1, keepdims=True)
    gate_ref[...] = 1.0 / ssum


_GATE_KW = dict(
    grid=(NB,),
    in_specs=[
        pl.BlockSpec((BLK, D), lambda i: (i, 0)),
        pl.BlockSpec((BLK, D), lambda i: (i, 0)),
        pl.BlockSpec((BLK, D), lambda i: (i, 0)),
        pl.BlockSpec((BLK, D), lambda i: (i, 0)),
        pl.BlockSpec((D, E), lambda i: (0, 0)),
    ],
    out_specs=[
        pl.BlockSpec((BLK, D), lambda i: (i, 0)),
        pl.BlockSpec((BLK, 1), lambda i: (i, 0)),
        pl.BlockSpec((BLK, 1), lambda i: (i, 0)),
    ],
    out_shape=[
        jax.ShapeDtypeStruct((S, D), jnp.float32),
        jax.ShapeDtypeStruct((S, 1), jnp.float32),
        jax.ShapeDtypeStruct((S, 1), jnp.float32),
    ],
)
_gate_call = pl.pallas_call(_gate_body, **_GATE_KW)


def _qkv_body(off_ref, xs_ref, ec_ref, w_ref, out_ref, acc_ref):
    i = pl.program_id(0)
    qlo = i * BLK
    qhi = qlo + BLK
    acc_ref[...] = jnp.zeros_like(acc_ref)
    x = xs_ref[...]
    ec = ec_ref[...]
    for e in range(E):
        @pl.when((off_ref[e] < qhi) & (off_ref[e + 1] > qlo))
        def _(e=e):
            xm = jnp.where(ec == float(e), x, 0.0)
            acc_ref[...] += jnp.dot(xm, w_ref[e], preferred_element_type=jnp.float32)
    out_ref[...] = acc_ref[...]


_QKV_KW = dict(
    grid=(NB,),
    in_specs=[
        pl.BlockSpec(memory_space=pltpu.SMEM),
        pl.BlockSpec((BLK, D), lambda i: (i, 0)),
        pl.BlockSpec((BLK, 1), lambda i: (i, 0)),
        pl.BlockSpec((E, D, F3), lambda i: (0, 0, 0)),
    ],
    out_specs=pl.BlockSpec((BLK, F3), lambda i: (i, 0)),
    out_shape=jax.ShapeDtypeStruct((S, F3), jnp.float32),
    scratch_shapes=[pltpu.VMEM((BLK, F3), jnp.float32)],
)
_qkv_call = pl.pallas_call(_qkv_body, **_QKV_KW)


def _attn_body(off_ref, q_ref, k_ref, v_ref, ec_ref, er_ref, gc_ref, wff_ref,
               bff_ref, out_ref, sc_ref, ctx_ref, acc_ref):
    i = pl.program_id(0)
    qlo = i * BLK
    qhi = qlo + BLK
    ov = []
    kmin = S
    kmax = 0
    for e in range(E):
        o0 = off_ref[e]
        o1 = off_ref[e + 1]
        ove = (o0 < qhi) & (o1 > qlo)
        ov.append(ove)
        kmin = jnp.where(ove, jnp.minimum(kmin, o0), kmin)
        kmax = jnp.where(ove, jnp.maximum(kmax, o1), kmax)

    sc_ref[...] = jnp.full((BLK, S), NEG, jnp.float32)
    q = q_ref[...]
    ec = ec_ref[...]
    for j in range(NB):
        klo = j * BLK

        @pl.when((klo < kmax) & (klo + BLK > kmin))
        def _(klo=klo):
            kj = k_ref[pl.ds(klo, BLK), :]
            s = lax.dot_general(q, kj, (((1,), (1,)), ((), ())),
                                preferred_element_type=jnp.float32) * SCALE
            msk = ec == er_ref[:, pl.ds(klo, BLK)]
            sc_ref[:, pl.ds(klo, BLK)] = jnp.where(msk, s, NEG)

    sall = sc_ref[...]
    mx = jnp.max(sall, axis=1, keepdims=True)
    p = jnp.exp(sall - mx)
    denom = jnp.sum(p, axis=1, keepdims=True)
    sc_ref[...] = p

    ctx_ref[...] = jnp.zeros_like(ctx_ref)
    for j in range(NB):
        klo = j * BLK

        @pl.when((klo < kmax) & (klo + BLK > kmin))
        def _(klo=klo):
            pj = sc_ref[:, pl.ds(klo, BLK)]
            ctx_ref[...] += jnp.dot(pj, v_ref[pl.ds(klo, BLK), :],
                                    preferred_element_type=jnp.float32)

    ctx = ctx_ref[...] / denom
    acc_ref[...] = jnp.zeros_like(acc_ref)
    for e in range(E):
        @pl.when(ov[e])
        def _(e=e):
            cm = jnp.where(ec == float(e), ctx, 0.0)
            acc_ref[...] += jnp.dot(cm, wff_ref[e], preferred_element_type=jnp.float32)
    out_ref[...] = (acc_ref[...] + bff_ref[...]) * gc_ref[...]


_ATTN_KW = dict(
    grid=(NB,),
    in_specs=[
        pl.BlockSpec(memory_space=pltpu.SMEM),
        pl.BlockSpec((BLK, HD), lambda i: (i, 0)),
        pl.BlockSpec((S, HD), lambda i: (0, 0)),
        pl.BlockSpec((S, HD), lambda i: (0, 0)),
        pl.BlockSpec((BLK, 1), lambda i: (i, 0)),
        pl.BlockSpec((1, S), lambda i: (0, 0)),
        pl.BlockSpec((BLK, 1), lambda i: (i, 0)),
        pl.BlockSpec((E, HD, D), lambda i: (0, 0, 0)),
        pl.BlockSpec((1, D), lambda i: (0, 0)),
    ],
    out_specs=pl.BlockSpec((BLK, D), lambda i: (i, 0)),
    out_shape=jax.ShapeDtypeStruct((S, D), jnp.float32),
    scratch_shapes=[
        pltpu.VMEM((BLK, S), jnp.float32),
        pltpu.VMEM((BLK, HD), jnp.float32),
        pltpu.VMEM((BLK, D), jnp.float32),
    ],
)
_attn_call = pl.pallas_call(_attn_body, **_ATTN_KW)


def _sc_gather(table, idx):
    NC, NS = 2, 16
    bpw = S // (NC * NS)
    mesh = plsc.VectorSubcoreMesh(core_axis_name="c", subcore_axis_name="s")

    @functools.partial(
        pl.kernel,
        out_type=jax.ShapeDtypeStruct((S, D), jnp.float32),
        mesh=mesh,
        scratch_types=[
            pltpu.VMEM((bpw,), jnp.int32),
            pltpu.VMEM((bpw, D), jnp.float32),
            pltpu.SemaphoreType.DMA,
        ],
    )
    def gk(table_hbm, idx_hbm, out_hbm, idx_v, rows_v, sem):
        wid = lax.axis_index("s") * NC + lax.axis_index("c")
        base = wid * bpw
        pltpu.sync_copy(idx_hbm.at[pl.ds(base, bpw)], idx_v)
        pltpu.async_copy(table_hbm.at[idx_v], rows_v, sem).wait()
        pltpu.sync_copy(rows_v, out_hbm.at[pl.ds(base, bpw)])

    return gk(table, idx)


def kernel(x, mask, Wg, Wqkv, Wff, bff):
    xf = x.reshape(S, D)
    xp1 = xf[:, _P1]
    xp2 = xf[:, _P2]
    x2, eidc, gatec = _gate_call(xp1, xp2, jnp.asarray(_AC), jnp.asarray(_BC), Wg)

    eid = eidc[:, 0]
    perm = jnp.argsort(eid, stable=True).astype(jnp.int32)
    inv = jnp.argsort(perm).astype(jnp.int32)
    eid_s = jnp.sort(eid)
    gate_s = gatec[:, 0][perm]
    offsets = jnp.searchsorted(
        eid_s, jnp.arange(E + 1, dtype=jnp.float32), side="left"
    ).astype(jnp.int32)

    xs = _sc_gather(x2, perm)
    qkv = _qkv_call(offsets, xs, eid_s[:, None], Wqkv)
    q = qkv[:, :HD]
    k = qkv[:, HD:2 * HD]
    v = qkv[:, 2 * HD:]
    outs = _attn_call(offsets, q, k, v, eid_s[:, None], eid_s[None, :],
                      gate_s[:, None], Wff, bff[None, :])
    out = _sc_gather(outs, inv)
    return out.reshape(1, S, D)


# speedup vs baseline: 4.5706x; 1.5006x over previous
import functools

import numpy as np
import jax
import jax.numpy as jnp
from jax import lax
from jax.experimental import pallas as pl
from jax.experimental.pallas import tpu as pltpu
from jax.experimental.pallas import tpu_sc as plsc

E = 8
S = 2048
D = 1024
HD = D // E
RP = HD // 2
F3 = 3 * HD
NH = 16
SCALE = 1.0 / float(np.sqrt(D // NH))
BLK = 256
NB = S // BLK
NEG = -1e30


def _rotary_tables():
    freqs = (1.0 / (10000.0 ** (np.arange(0, RP, 2, dtype=np.float32) / np.float32(RP)))).astype(np.float32)
    ang = np.arange(S, dtype=np.float32)[:, None] * freqs[None, :]
    cos = np.cos(ang).astype(np.float32)
    sin = np.sin(ang).astype(np.float32)
    Q1 = np.zeros((HD, HD), dtype=np.float32)
    Q2 = np.zeros((HD, HD), dtype=np.float32)
    A = np.zeros((S, HD), dtype=np.float32)
    B = np.zeros((S, HD), dtype=np.float32)
    for o in range(HD):
        if o < RP:
            j = o // 2
            Q1[RP + 2 * j, o] = 1.0
            Q2[RP + 2 * j + 1, o] = 1.0
            if o % 2 == 0:
                A[:, o] = cos[:, j]
                B[:, o] = -sin[:, j]
            else:
                A[:, o] = sin[:, j]
                B[:, o] = cos[:, j]
        else:
            Q1[o - RP, o] = 1.0
            A[:, o] = 1.0
    return Q1, Q2, A, B


_Q1, _Q2, _AC, _BC = _rotary_tables()


def _gate_body(x_ref, q1_ref, q2_ref, ac_ref, bc_ref, wg_ref, x2_ref, eid_ref, gate_ref):
    x4 = x_ref[...].reshape(BLK * E, HD)
    y1 = jnp.dot(x4, q1_ref[...], preferred_element_type=jnp.float32).reshape(BLK, E, HD)
    y2 = jnp.dot(x4, q2_ref[...], preferred_element_type=jnp.float32).reshape(BLK, E, HD)
    a3 = ac_ref[...][:, None, :]
    b3 = bc_ref[...][:, None, :]
    x2 = (y1 * a3 + y2 * b3).reshape(BLK, D)
    x2_ref[...] = x2
    logits = jnp.dot(x2, wg_ref[...], preferred_element_type=jnp.float32)
    m = jnp.max(logits, axis=1, keepdims=True)
    p = jnp.exp(logits - m)
    ssum = jnp.sum(p, axis=1, keepdims=True)
    probs = p / ssum
    pmax = jnp.max(probs, axis=1, keepdims=True)
    ids = lax.broadcasted_iota(jnp.int32, (BLK, E), 1).astype(jnp.float32)
    eid_ref[...] = jnp.min(jnp.where(probs >= pmax, ids, float(E)), axis=1, keepdims=True)
    gate_ref[...] = 1.0 / ssum


_GATE_KW = dict(
    grid=(NB,),
    in_specs=[
        pl.BlockSpec((BLK, D), lambda i: (i, 0)),
        pl.BlockSpec((HD, HD), lambda i: (0, 0)),
        pl.BlockSpec((HD, HD), lambda i: (0, 0)),
        pl.BlockSpec((BLK, HD), lambda i: (i, 0)),
        pl.BlockSpec((BLK, HD), lambda i: (i, 0)),
        pl.BlockSpec((D, E), lambda i: (0, 0)),
    ],
    out_specs=[
        pl.BlockSpec((BLK, D), lambda i: (i, 0)),
        pl.BlockSpec((BLK, 1), lambda i: (i, 0)),
        pl.BlockSpec((BLK, 1), lambda i: (i, 0)),
    ],
    out_shape=[
        jax.ShapeDtypeStruct((S, D), jnp.float32),
        jax.ShapeDtypeStruct((S, 1), jnp.float32),
        jax.ShapeDtypeStruct((S, 1), jnp.float32),
    ],
)
_gate_call = pl.pallas_call(_gate_body, **_GATE_KW)


def _qkv_body(off_ref, xs_ref, ec_ref, w_ref, q_ref, k_ref, v_ref, acc_ref):
    i = pl.program_id(0)
    qlo = i * BLK
    qhi = qlo + BLK
    acc_ref[...] = jnp.zeros_like(acc_ref)
    x = xs_ref[...]
    ec = ec_ref[...]
    for e in range(E):
        @pl.when((off_ref[e] < qhi) & (off_ref[e + 1] > qlo))
        def _(e=e):
            xm = jnp.where(ec == float(e), x, 0.0)
            acc_ref[...] += jnp.dot(xm, w_ref[e], preferred_element_type=jnp.float32)
    acc = acc_ref[...]
    q_ref[...] = acc[:, :HD]
    k_ref[...] = acc[:, HD:2 * HD]
    v_ref[...] = acc[:, 2 * HD:]


_QKV_KW = dict(
    grid=(NB,),
    in_specs=[
        pl.BlockSpec(memory_space=pltpu.SMEM),
        pl.BlockSpec((BLK, D), lambda i: (i, 0)),
        pl.BlockSpec((BLK, 1), lambda i: (i, 0)),
        pl.BlockSpec((E, D, F3), lambda i: (0, 0, 0)),
    ],
    out_specs=[
        pl.BlockSpec((BLK, HD), lambda i: (i, 0)),
        pl.BlockSpec((BLK, HD), lambda i: (i, 0)),
        pl.BlockSpec((BLK, HD), lambda i: (i, 0)),
    ],
    out_shape=[
        jax.ShapeDtypeStruct((S, HD), jnp.float32),
        jax.ShapeDtypeStruct((S, HD), jnp.float32),
        jax.ShapeDtypeStruct((S, HD), jnp.float32),
    ],
    scratch_shapes=[pltpu.VMEM((BLK, F3), jnp.float32)],
)
_qkv_call = pl.pallas_call(_qkv_body, **_QKV_KW)


def _attn_body(off_ref, q_ref, k_ref, v_ref, ec_ref, er_ref, gc_ref, wff_ref,
               bff_ref, out_ref, sc_ref, ctx_ref, acc_ref):
    i = pl.program_id(0)
    qlo = i * BLK
    qhi = qlo + BLK
    ov = []
    kmin = S
    kmax = 0
    for e in range(E):
        o0 = off_ref[e]
        o1 = off_ref[e + 1]
        ove = (o0 < qhi) & (o1 > qlo)
        ov.append(ove)
        kmin = jnp.where(ove, jnp.minimum(kmin, o0), kmin)
        kmax = jnp.where(ove, jnp.maximum(kmax, o1), kmax)

    sc_ref[...] = jnp.full((BLK, S), NEG, jnp.float32)
    q = q_ref[...]
    ec = ec_ref[...]
    for j in range(NB):
        klo = j * BLK

        @pl.when((klo < kmax) & (klo + BLK > kmin))
        def _(klo=klo):
            kj = k_ref[pl.ds(klo, BLK), :]
            s = lax.dot_general(q, kj, (((1,), (1,)), ((), ())),
                                preferred_element_type=jnp.float32) * SCALE
            msk = ec == er_ref[:, pl.ds(klo, BLK)]
            sc_ref[:, pl.ds(klo, BLK)] = jnp.where(msk, s, NEG)

    sall = sc_ref[...]
    mx = jnp.max(sall, axis=1, keepdims=True)
    p = jnp.exp(sall - mx)
    denom = jnp.sum(p, axis=1, keepdims=True)
    sc_ref[...] = p

    ctx_ref[...] = jnp.zeros_like(ctx_ref)
    for j in range(NB):
        klo = j * BLK

        @pl.when((klo < kmax) & (klo + BLK > kmin))
        def _(klo=klo):
            pj = sc_ref[:, pl.ds(klo, BLK)]
            ctx_ref[...] += jnp.dot(pj, v_ref[pl.ds(klo, BLK), :],
                                    preferred_element_type=jnp.float32)

    ctx = ctx_ref[...] / denom
    acc_ref[...] = jnp.zeros_like(acc_ref)
    for e in range(E):
        @pl.when(ov[e])
        def _(e=e):
            cm = jnp.where(ec == float(e), ctx, 0.0)
            acc_ref[...] += jnp.dot(cm, wff_ref[e], preferred_element_type=jnp.float32)
    out_ref[...] = (acc_ref[...] + bff_ref[...]) * gc_ref[...]


_ATTN_KW = dict(
    grid=(NB,),
    in_specs=[
        pl.BlockSpec(memory_space=pltpu.SMEM),
        pl.BlockSpec((BLK, HD), lambda i: (i, 0)),
        pl.BlockSpec((S, HD), lambda i: (0, 0)),
        pl.BlockSpec((S, HD), lambda i: (0, 0)),
        pl.BlockSpec((BLK, 1), lambda i: (i, 0)),
        pl.BlockSpec((1, S), lambda i: (0, 0)),
        pl.BlockSpec((BLK, 1), lambda i: (i, 0)),
        pl.BlockSpec((E, HD, D), lambda i: (0, 0, 0)),
        pl.BlockSpec((1, D), lambda i: (0, 0)),
    ],
    out_specs=pl.BlockSpec((BLK, D), lambda i: (i, 0)),
    out_shape=jax.ShapeDtypeStruct((S, D), jnp.float32),
    scratch_shapes=[
        pltpu.VMEM((BLK, S), jnp.float32),
        pltpu.VMEM((BLK, HD), jnp.float32),
        pltpu.VMEM((BLK, D), jnp.float32),
    ],
)
_attn_call = pl.pallas_call(_attn_body, **_ATTN_KW)


def _sc_gather(table, idx):
    NC, NS = 2, 16
    bpw = S // (NC * NS)
    mesh = plsc.VectorSubcoreMesh(core_axis_name="c", subcore_axis_name="s")

    @functools.partial(
        pl.kernel,
        out_type=jax.ShapeDtypeStruct((S, D), jnp.float32),
        mesh=mesh,
        scratch_types=[
            pltpu.VMEM((bpw,), jnp.int32),
            pltpu.VMEM((bpw, D), jnp.float32),
            pltpu.SemaphoreType.DMA,
        ],
    )
    def gk(table_hbm, idx_hbm, out_hbm, idx_v, rows_v, sem):
        wid = lax.axis_index("s") * NC + lax.axis_index("c")
        base = wid * bpw
        pltpu.sync_copy(idx_hbm.at[pl.ds(base, bpw)], idx_v)
        pltpu.async_copy(table_hbm.at[idx_v], rows_v, sem).wait()
        pltpu.sync_copy(rows_v, out_hbm.at[pl.ds(base, bpw)])

    return gk(table, idx)


def kernel(x, mask, Wg, Wqkv, Wff, bff):
    xf = x.reshape(S, D)
    x2, eidc, gatec = _gate_call(xf, jnp.asarray(_Q1), jnp.asarray(_Q2),
                                 jnp.asarray(_AC), jnp.asarray(_BC), Wg)

    eid = eidc[:, 0]
    perm = jnp.argsort(eid, stable=True).astype(jnp.int32)
    iot = jnp.arange(S, dtype=jnp.int32)
    inv = jnp.zeros((S,), jnp.int32).at[perm].set(iot)
    counts = jnp.sum(eid[:, None] == jnp.arange(E, dtype=jnp.float32)[None, :],
                     axis=0, dtype=jnp.int32)
    offsets = jnp.concatenate([jnp.zeros((1,), jnp.int32), jnp.cumsum(counts)])
    eid_s = jnp.sum(iot[:, None] >= offsets[None, 1:E + 1],
                    axis=1, dtype=jnp.int32).astype(jnp.float32)
    gate_s = jnp.take(gatec[:, 0], perm)

    xs = _sc_gather(x2, perm)
    q, k, v = _qkv_call(offsets, xs, eid_s[:, None], Wqkv)
    outs = _attn_call(offsets, q, k, v, eid_s[:, None], eid_s[None, :],
                      gate_s[:, None], Wff, bff[None, :])
    out = _sc_gather(outs, inv)
    return out.reshape(1, S, D)
